# baseline (speedup 1.0000x reference)
import jax
import jax.numpy as jnp
from jax import lax
from jax.experimental import pallas as pl
from jax.experimental.pallas import tpu as pltpu
from jax.experimental.pallas import tpu_sc as plsc

NC, NS = 2, 16
NW = NC * NS
CH = 128
D = 64
H = 32
LANES = 16


def _dot_rows(a_ref, b_ref, out_ref, nrows):
    iota = lax.iota(jnp.int32, LANES)

    def body(g, carry):
        r = g * LANES + iota
        accs = [jnp.zeros((LANES,), jnp.float32) for _ in range(4)]
        for d in range(D):
            dv = jnp.full((LANES,), d, jnp.int32)
            a = plsc.load_gather(a_ref, [r, dv])
            b = plsc.load_gather(b_ref, [r, dv])
            accs[d % 4] = accs[d % 4] + a * b
        out_ref[pl.ds(g * LANES, LANES)] = (accs[0] + accs[1]) + (accs[2] + accs[3])
        return carry

    lax.fori_loop(0, nrows // LANES, body, 0)


def _make_sc_kernel(B):
    bpw = B // NW
    nch = bpw // CH

    mesh = plsc.VectorSubcoreMesh(
        core_axis_name="c", subcore_axis_name="s",
        num_cores=NC, num_subcores=NS)

    def body(su_t, tu_t, si_t, ti_t,
             su_i, si_i, tu_i, ti_i, mu_i,
             ss_o, ts_o, se_o, te_o,
             idx_a, idx_b, rows_a, rows_b, sc_v, sem_a, sem_b):
        wid = lax.axis_index("s") * NC + lax.axis_index("c")
        base = wid * bpw

        def gather(table, idx_ref, dst_ref, sem):
            cps = []
            for c in range(nch):
                cps.append(pltpu.async_copy(
                    table.at[idx_ref.at[c]],
                    dst_ref.at[pl.ds(c * CH, CH)], sem))
            return cps

        pltpu.sync_copy(mu_i.at[wid], idx_a)
        cps = gather(su_t, idx_a, rows_a, sem_a)
        cps += gather(tu_t, idx_a, rows_b, sem_b)
        for cp in cps:
            cp.wait()
        pltpu.sync_copy(rows_a, se_o.at[pl.ds(base, bpw)])
        pltpu.sync_copy(rows_b, te_o.at[pl.ds(base, bpw)])

        pltpu.sync_copy(su_i.at[wid], idx_a)
        pltpu.sync_copy(si_i.at[wid], idx_b)
        cps = gather(su_t, idx_a, rows_a, sem_a)
        cps += gather(si_t, idx_b, rows_b, sem_b)
        for cp in cps:
            cp.wait()
        _dot_rows(rows_a, rows_b, sc_v, bpw)
        pltpu.sync_copy(sc_v, ss_o.at[pl.ds(base, bpw)])

        pltpu.sync_copy(tu_i.at[wid], idx_a)
        pltpu.sync_copy(ti_i.at[wid], idx_b)
        cps = gather(tu_t, idx_a, rows_a, sem_a)
        cps += gather(ti_t, idx_b, rows_b, sem_b)
        for cp in cps:
            cp.wait()
        _dot_rows(rows_a, rows_b, sc_v, bpw)
        pltpu.sync_copy(sc_v, ts_o.at[pl.ds(base, bpw)])

    return pl.kernel(
        body,
        out_type=(
            jax.ShapeDtypeStruct((B,), jnp.float32),
            jax.ShapeDtypeStruct((B,), jnp.float32),
            jax.ShapeDtypeStruct((B, D), jnp.float32),
            jax.ShapeDtypeStruct((B, D), jnp.float32),
        ),
        mesh=mesh,
        scratch_types=[
            pltpu.VMEM((nch, CH), jnp.int32),
            pltpu.VMEM((nch, CH), jnp.int32),
            pltpu.VMEM((bpw, D), jnp.float32),
            pltpu.VMEM((bpw, D), jnp.float32),
            pltpu.VMEM((bpw,), jnp.float32),
            pltpu.SemaphoreType.DMA,
            pltpu.SemaphoreType.DMA,
        ],
        compiler_params=pltpu.CompilerParams(
            needs_layout_passes=False, use_tc_tiling_on_sc=False),
    )


def _mlp_body(x_ref, wi_ref, bi_ref, wh_ref, bh_ref, wo_ref, bo_ref, o_ref):
    x = x_ref[...]
    h = jnp.maximum(
        jnp.dot(x, wi_ref[...], preferred_element_type=jnp.float32) + bi_ref[...],
        0.0)
    for _ in range(2):
        h = jnp.maximum(
            jnp.dot(h, wh_ref[...], preferred_element_type=jnp.float32) + bh_ref[...],
            0.0)
    o_ref[...] = (
        jnp.dot(h, wo_ref[...], preferred_element_type=jnp.float32) + bo_ref[...])


def _mlp(x, W_in, b_in, W_hid, b_hid, W_out, b_out):
    B = x.shape[0]
    blk = 2048
    grid = (B // blk,)
    full = lambda shape: pl.BlockSpec(shape, lambda i: (0, 0))
    return pl.pallas_call(
        _mlp_body,
        grid=grid,
        in_specs=[
            pl.BlockSpec((blk, D), lambda i: (i, 0)),
            full((D, H)), full((1, H)),
            full((H, H)), full((1, H)),
            full((H, D)), full((1, D)),
        ],
        out_specs=pl.BlockSpec((blk, D), lambda i: (i, 0)),
        out_shape=jax.ShapeDtypeStruct((B, D), jnp.float32),
    )(x, W_in, b_in.reshape(1, H), W_hid, b_hid.reshape(1, H),
      W_out, b_out.reshape(1, D))


def kernel(suser, sitem, tuser, titem, muser,
           su_table, tu_table, si_table, ti_table,
           W_in, b_in, W_hid, b_hid, W_out, b_out):
    B = suser.shape[0]
    bpw = B // NW
    nch = bpw // CH
    shp = (NW, nch, CH)
    sus, sis, tus, tis, mus = (
        a.reshape(shp) for a in (suser, sitem, tuser, titem, muser))
    sc = _make_sc_kernel(B)
    sscore, tscore, su_emb, tu_emb = sc(
        su_table, tu_table, si_table, ti_table, sus, sis, tus, tis, mus)
    f_su = _mlp(su_emb, W_in, b_in, W_hid, b_hid, W_out, b_out)
    return (sscore, tscore, f_su, tu_emb)


# speedup vs baseline: 1.0405x; 1.0405x over previous
import jax
import jax.numpy as jnp
from jax import lax
from jax.experimental.layout import Format, Layout, with_layout_constraint
from jax.experimental import pallas as pl
from jax.experimental.pallas import tpu as pltpu
from jax.experimental.pallas import tpu_sc as plsc

NC, NS = 2, 16
NW = NC * NS
D = 64
H = 32
LANES = 16
FCH = 128


def _fetch_rows(tbl, idx_v, buf, sem, lo, n):
    def body(g, carry):
        v = idx_v[pl.ds(g * LANES, LANES)]
        for k in range(LANES):
            i = v[k]
            j = g * LANES + k
            pltpu.async_copy(
                tbl.at[pl.ds(i * D, D)], buf.at[pl.ds(j * D, D)], sem)
        return carry

    lax.fori_loop(lo // LANES, (lo + n) // LANES, body, 0)


def _drain(dummy_hbm, buf, sem, lo, n):
    pltpu.make_async_copy(
        dummy_hbm.at[pl.ds(0, n * D)], buf.at[pl.ds(lo * D, n * D)], sem
    ).wait()


def _gather_pair(ta, ia, bufa, sema, tb, ib, bufb, semb, dummy, bpw):
    nch = bpw // FCH
    for c in range(nch):
        _fetch_rows(ta, ia, bufa, sema, c * FCH, FCH)
        _fetch_rows(tb, ib, bufb, semb, c * FCH, FCH)
        if c:
            _drain(dummy, bufa, sema, (c - 1) * FCH, FCH)
            _drain(dummy, bufb, semb, (c - 1) * FCH, FCH)
    _drain(dummy, bufa, sema, (nch - 1) * FCH, FCH)
    _drain(dummy, bufb, semb, (nch - 1) * FCH, FCH)


def _dot_rows(a_ref, b_ref, strip, out_v, n):
    rows16 = lax.iota(jnp.int32, LANES) * LANES

    def body(g, carry):
        for r in range(LANES):
            j = g * LANES + r
            s = jnp.zeros((LANES,), jnp.float32)
            for q in range(D // LANES):
                a = a_ref[pl.ds(j * D + q * LANES, LANES)]
                b = b_ref[pl.ds(j * D + q * LANES, LANES)]
                s = s + a * b
            strip[pl.ds(r * LANES, LANES)] = s
        acc = jnp.zeros((LANES,), jnp.float32)
        for l in range(LANES):
            acc = acc + plsc.load_gather(strip, [rows16 + l])
        out_v[pl.ds(g * LANES, LANES)] = acc
        return carry

    lax.fori_loop(0, n // LANES, body, 0)


def _make_sc_kernel(B):
    bpw = B // NW

    mesh = plsc.VectorSubcoreMesh(
        core_axis_name="c", subcore_axis_name="s",
        num_cores=NC, num_subcores=NS)

    def body(su_t, tu_t, si_t, ti_t,
             su_i, si_i, tu_i, ti_i, mu_i,
             ss_o, ts_o, se_o, te_o,
             idx_va, idx_vb, strip, sc_v,
             buf_a, buf_b, sem_a, sem_b):
        wid = lax.axis_index("s") * NC + lax.axis_index("c")
        base = wid * bpw

        def load_idx(src_hbm, vmem):
            pltpu.sync_copy(src_hbm.at[pl.ds(base, bpw)], vmem)

        load_idx(mu_i, idx_va)
        _gather_pair(su_t, idx_va, buf_a, sem_a,
                     tu_t, idx_va, buf_b, sem_b, se_o, bpw)
        pltpu.sync_copy(buf_a, se_o.at[pl.ds(base * D, bpw * D)])
        pltpu.sync_copy(buf_b, te_o.at[pl.ds(base * D, bpw * D)])

        load_idx(su_i, idx_va)
        load_idx(si_i, idx_vb)
        _gather_pair(su_t, idx_va, buf_a, sem_a,
                     si_t, idx_vb, buf_b, sem_b, se_o, bpw)
        _dot_rows(buf_a, buf_b, strip, sc_v, bpw)
        pltpu.sync_copy(sc_v, ss_o.at[pl.ds(base, bpw)])

        load_idx(tu_i, idx_va)
        load_idx(ti_i, idx_vb)
        _gather_pair(tu_t, idx_va, buf_a, sem_a,
                     ti_t, idx_vb, buf_b, sem_b, se_o, bpw)
        _dot_rows(buf_a, buf_b, strip, sc_v, bpw)
        pltpu.sync_copy(sc_v, ts_o.at[pl.ds(base, bpw)])

    return pl.kernel(
        body,
        out_type=(
            jax.ShapeDtypeStruct((B,), jnp.float32),
            jax.ShapeDtypeStruct((B,), jnp.float32),
            jax.ShapeDtypeStruct((B * D,), jnp.float32),
            jax.ShapeDtypeStruct((B * D,), jnp.float32),
        ),
        mesh=mesh,
        scratch_types=[
            pltpu.VMEM((bpw,), jnp.int32),
            pltpu.VMEM((bpw,), jnp.int32),
            pltpu.VMEM((LANES * LANES,), jnp.float32),
            pltpu.VMEM((bpw,), jnp.float32),
            pltpu.VMEM((bpw * D,), jnp.float32),
            pltpu.VMEM((bpw * D,), jnp.float32),
            pltpu.SemaphoreType.DMA,
            pltpu.SemaphoreType.DMA,
        ],
        compiler_params=pltpu.CompilerParams(needs_layout_passes=False),
    )


def _mlp_body(x_ref, wi_ref, bi_ref, wh_ref, bh_ref, wo_ref, bo_ref, o_ref):
    x = x_ref[...]
    h = jnp.maximum(
        jnp.dot(x, wi_ref[...], preferred_element_type=jnp.float32)
        + bi_ref[...], 0.0)
    for _ in range(2):
        h = jnp.maximum(
            jnp.dot(h, wh_ref[...], preferred_element_type=jnp.float32)
            + bh_ref[...], 0.0)
    o_ref[...] = (
        jnp.dot(h, wo_ref[...], preferred_element_type=jnp.float32)
        + bo_ref[...])


def _mlp(x, W_in, b_in, W_hid, b_hid, W_out, b_out):
    B = x.shape[0]
    blk = 2048
    grid = (B // blk,)
    full = lambda shape: pl.BlockSpec(shape, lambda i: (0, 0))
    return pl.pallas_call(
        _mlp_body,
        grid=grid,
        in_specs=[
            pl.BlockSpec((blk, D), lambda i: (i, 0)),
            full((D, H)), full((1, H)),
            full((H, H)), full((1, H)),
            full((H, D)), full((1, D)),
        ],
        out_specs=pl.BlockSpec((blk, D), lambda i: (i, 0)),
        out_shape=jax.ShapeDtypeStruct((B, D), jnp.float32),
    )(x, W_in, b_in.reshape(1, H), W_hid, b_hid.reshape(1, H),
      W_out, b_out.reshape(1, D))


def _impl(suser, sitem, tuser, titem, muser,
          su_table, tu_table, si_table, ti_table,
          W_in, b_in, W_hid, b_hid, W_out, b_out):
    B = suser.shape[0]
    sc = _make_sc_kernel(B)
    sscore, tscore, se, te = sc(
        su_table.reshape(-1), tu_table.reshape(-1),
        si_table.reshape(-1), ti_table.reshape(-1),
        suser, sitem, tuser, titem, muser)
    f = _mlp(se.reshape(B, D), W_in, b_in, W_hid, b_hid, W_out, b_out)
    return (sscore, tscore, f, te.reshape(B, D))


_jit_impl_cache = {}


def _jit_impl_get():
    try:
        dev = jax.devices("tpu")[0]
    except RuntimeError:
        dev = jax.devices()[0]
    fn = _jit_impl_cache.get(dev)
    if fn is None:
        rm = Format(Layout((0, 1)), jax.sharding.SingleDeviceSharding(dev))
        fmts = ([None] * 5) + ([rm] * 4) + ([None] * 6)
        fn = jax.jit(_impl, in_shardings=fmts)
        _jit_impl_cache[dev] = fn
    return fn


def kernel(*args):
    try:
        return _jit_impl_get()(*args)
    except ValueError:
        return _impl(*args)
